# baseline (device time: 261360 ns/iter reference)
import jax
import jax.numpy as jnp
from jax import lax
from jax.experimental import pallas as pl
from jax.experimental.pallas import tpu as pltpu

N_DEV = 16


def kernel(A, B):
    m, k = A.shape
    _, n = B.shape
    rows = m // N_DEV

    def body(a_ref, b_ref, out_ref, acc_ref, rs_buf,
             rs_send, rs_recv, ag_send, ag_recv):
        my = lax.axis_index("i")
        left = lax.rem(my + N_DEV - 1, N_DEV)
        right = lax.rem(my + 1, N_DEV)

        barrier = pltpu.get_barrier_semaphore()
        for nbr in (left, right):
            pl.semaphore_signal(barrier, inc=1, device_id=(nbr,),
                                device_id_type=pl.DeviceIdType.MESH)
        pl.semaphore_wait(barrier, 2)

        a = a_ref[...].astype(jnp.bfloat16)
        b = b_ref[...].astype(jnp.bfloat16)
        acc_ref[...] = jnp.dot(a, b, preferred_element_type=jnp.float32)

        for s in range(N_DEV - 1):
            send_idx = lax.rem(my - s + N_DEV, N_DEV)
            recv_idx = lax.rem(my - s - 1 + N_DEV, N_DEV)
            rdma = pltpu.make_async_remote_copy(
                src_ref=acc_ref.at[pl.ds(send_idx * rows, rows), :],
                dst_ref=rs_buf.at[s],
                send_sem=rs_send.at[s],
                recv_sem=rs_recv.at[s],
                device_id=(right,),
                device_id_type=pl.DeviceIdType.MESH,
            )
            rdma.start()
            rdma.wait()
            acc_ref[pl.ds(recv_idx * rows, rows), :] = (
                acc_ref[pl.ds(recv_idx * rows, rows), :] + rs_buf[s]
            )

        own = lax.rem(my + 1, N_DEV)
        out_ref[pl.ds(own * rows, rows), :] = jnp.maximum(
            acc_ref[pl.ds(own * rows, rows), :], 0.0
        )

        for s in range(N_DEV - 1):
            send_idx = lax.rem(my + 1 - s + N_DEV, N_DEV)
            rdma = pltpu.make_async_remote_copy(
                src_ref=out_ref.at[pl.ds(send_idx * rows, rows), :],
                dst_ref=out_ref.at[pl.ds(send_idx * rows, rows), :],
                send_sem=ag_send.at[s],
                recv_sem=ag_recv.at[s],
                device_id=(right,),
                device_id_type=pl.DeviceIdType.MESH,
            )
            rdma.start()
            rdma.wait()

    return pl.pallas_call(
        body,
        out_shape=jax.ShapeDtypeStruct((m, n), jnp.float32),
        in_specs=[
            pl.BlockSpec(memory_space=pltpu.VMEM),
            pl.BlockSpec(memory_space=pltpu.VMEM),
        ],
        out_specs=pl.BlockSpec(memory_space=pltpu.VMEM),
        scratch_shapes=[
            pltpu.VMEM((m, n), jnp.float32),
            pltpu.VMEM((N_DEV - 1, rows, n), jnp.float32),
            pltpu.SemaphoreType.DMA((N_DEV - 1,)),
            pltpu.SemaphoreType.DMA((N_DEV - 1,)),
            pltpu.SemaphoreType.DMA((N_DEV - 1,)),
            pltpu.SemaphoreType.DMA((N_DEV - 1,)),
        ],
        compiler_params=pltpu.CompilerParams(collective_id=0),
    )(A, B)


# device time: 158631 ns/iter; 1.6476x vs baseline; 1.6476x over previous
import jax
import jax.numpy as jnp
from jax import lax
from jax.experimental import pallas as pl
from jax.experimental.pallas import tpu as pltpu

N_DEV = 16


def kernel(A, B):
    m, k = A.shape
    _, n = B.shape
    rows = m // N_DEV
    hn = n // 2

    def body(a_ref, b_ref, out_ref, acc_ref,
             rs_stage_r, rs_stage_l, rs_buf_r, rs_buf_l,
             ag_buf_r, ag_buf_l,
             rs_send_r, rs_recv_r, rs_send_l, rs_recv_l,
             ag_send_r, ag_recv_r, ag_send_l, ag_recv_l):
        my = lax.axis_index("i")
        left = lax.rem(my + N_DEV - 1, N_DEV)
        right = lax.rem(my + 1, N_DEV)

        barrier = pltpu.get_barrier_semaphore()
        for nbr in (left, right):
            pl.semaphore_signal(barrier, inc=1, device_id=(nbr,),
                                device_id_type=pl.DeviceIdType.MESH)
        pl.semaphore_wait(barrier, 2)

        a = a_ref[...].astype(jnp.bfloat16)
        b = b_ref[...].astype(jnp.bfloat16)
        acc_ref[...] = jnp.dot(a, b, preferred_element_type=jnp.float32)

        def rdma(src, dst, ssem, rsem, tgt):
            cp = pltpu.make_async_remote_copy(
                src_ref=src, dst_ref=dst, send_sem=ssem, recv_sem=rsem,
                device_id=(tgt,), device_id_type=pl.DeviceIdType.MESH,
            )
            cp.start()
            return cp

        for s in range(N_DEV - 1):
            send_r = lax.rem(my - s + N_DEV, N_DEV)
            recv_r = lax.rem(my - s - 1 + N_DEV, N_DEV)
            send_l = lax.rem(my + s, N_DEV)
            recv_l = lax.rem(my + s + 1, N_DEV)
            rs_stage_r[s] = acc_ref[
                pl.ds(send_r * rows, rows), pl.ds(0, hn)
            ].astype(jnp.bfloat16)
            rs_stage_l[s] = acc_ref[
                pl.ds(send_l * rows, rows), pl.ds(hn, hn)
            ].astype(jnp.bfloat16)
            cp_r = rdma(rs_stage_r.at[s], rs_buf_r.at[s],
                        rs_send_r.at[s], rs_recv_r.at[s], right)
            cp_l = rdma(rs_stage_l.at[s], rs_buf_l.at[s],
                        rs_send_l.at[s], rs_recv_l.at[s], left)
            cp_r.wait()
            cp_l.wait()
            acc_ref[pl.ds(recv_r * rows, rows), pl.ds(0, hn)] = (
                acc_ref[pl.ds(recv_r * rows, rows), pl.ds(0, hn)]
                + rs_buf_r[s].astype(jnp.float32)
            )
            acc_ref[pl.ds(recv_l * rows, rows), pl.ds(hn, hn)] = (
                acc_ref[pl.ds(recv_l * rows, rows), pl.ds(hn, hn)]
                + rs_buf_l[s].astype(jnp.float32)
            )

        own_r = lax.rem(my + 1, N_DEV)
        own_l = lax.rem(my + N_DEV - 1, N_DEV)
        ag_buf_r[0] = jnp.maximum(
            acc_ref[pl.ds(own_r * rows, rows), pl.ds(0, hn)], 0.0
        ).astype(jnp.bfloat16)
        ag_buf_l[0] = jnp.maximum(
            acc_ref[pl.ds(own_l * rows, rows), pl.ds(hn, hn)], 0.0
        ).astype(jnp.bfloat16)
        out_ref[pl.ds(own_r * rows, rows), pl.ds(0, hn)] = ag_buf_r[0]
        out_ref[pl.ds(own_l * rows, rows), pl.ds(hn, hn)] = ag_buf_l[0]

        for s in range(N_DEV - 1):
            got_r = lax.rem(my - s + N_DEV, N_DEV)
            got_l = lax.rem(my + s, N_DEV)
            cp_r = rdma(ag_buf_r.at[s], ag_buf_r.at[s + 1],
                        ag_send_r.at[s], ag_recv_r.at[s], right)
            cp_l = rdma(ag_buf_l.at[s], ag_buf_l.at[s + 1],
                        ag_send_l.at[s], ag_recv_l.at[s], left)
            cp_r.wait()
            cp_l.wait()
            out_ref[pl.ds(got_r * rows, rows), pl.ds(0, hn)] = ag_buf_r[s + 1]
            out_ref[pl.ds(got_l * rows, rows), pl.ds(hn, hn)] = ag_buf_l[s + 1]

    return pl.pallas_call(
        body,
        out_shape=jax.ShapeDtypeStruct((m, n), jnp.bfloat16),
        in_specs=[
            pl.BlockSpec(memory_space=pltpu.VMEM),
            pl.BlockSpec(memory_space=pltpu.VMEM),
        ],
        out_specs=pl.BlockSpec(memory_space=pltpu.VMEM),
        scratch_shapes=[
            pltpu.VMEM((m, n), jnp.float32),
            pltpu.VMEM((N_DEV - 1, rows, hn), jnp.bfloat16),
            pltpu.VMEM((N_DEV - 1, rows, hn), jnp.bfloat16),
            pltpu.VMEM((N_DEV - 1, rows, hn), jnp.bfloat16),
            pltpu.VMEM((N_DEV - 1, rows, hn), jnp.bfloat16),
            pltpu.VMEM((N_DEV, rows, hn), jnp.bfloat16),
            pltpu.VMEM((N_DEV, rows, hn), jnp.bfloat16),
            pltpu.SemaphoreType.DMA((N_DEV - 1,)),
            pltpu.SemaphoreType.DMA((N_DEV - 1,)),
            pltpu.SemaphoreType.DMA((N_DEV - 1,)),
            pltpu.SemaphoreType.DMA((N_DEV - 1,)),
            pltpu.SemaphoreType.DMA((N_DEV - 1,)),
            pltpu.SemaphoreType.DMA((N_DEV - 1,)),
            pltpu.SemaphoreType.DMA((N_DEV - 1,)),
            pltpu.SemaphoreType.DMA((N_DEV - 1,)),
        ],
        compiler_params=pltpu.CompilerParams(collective_id=0),
    )(A, B)


# device time: 98289 ns/iter; 2.6591x vs baseline; 1.6139x over previous
import functools

import jax
import jax.numpy as jnp
from jax import lax
from jax.experimental import pallas as pl
from jax.experimental.pallas import tpu as pltpu

N_DEV = 16
PLANE = 4
NZ = 4


def kernel(A, B):
    m, k = A.shape
    _, n = B.shape
    hn = n // 2
    qr_rows = m // PLANE
    h2 = qr_rows // 2
    h4 = qr_rows // 4

    def body(a_ref, b_ref, out_ref, acc_ref,
             p1_stage_r, p1_stage_l, p1_buf_r, p1_buf_l,
             sa_stage, sa_recv, sb_stage, sb_recv,
             pb_r, pb_l, agb_r, agb_l,
             p1_send_r, p1_recv_r, p1_send_l, p1_recv_l,
             p2_send, p2_recv,
             p3_send_r, p3_recv_r, p3_send_l, p3_recv_l):
        my = lax.axis_index("i")
        z = lax.div(my, PLANE)
        q = lax.rem(my, PLANE)
        qr = z * PLANE + lax.rem(q + 1, PLANE)
        ql = z * PLANE + lax.rem(q + PLANE - 1, PLANE)
        pz1 = q + PLANE * jnp.bitwise_xor(z, 1)
        pz2 = q + PLANE * jnp.bitwise_xor(z, 2)
        neighbors = (qr, ql, pz1, pz2)

        barrier = pltpu.get_barrier_semaphore()
        for nbr in neighbors:
            pl.semaphore_signal(barrier, inc=1, device_id=(nbr,),
                                device_id_type=pl.DeviceIdType.MESH)
        pl.semaphore_wait(barrier, len(neighbors))

        a = a_ref[...].astype(jnp.bfloat16)
        b = b_ref[...].astype(jnp.bfloat16)
        acc_ref[...] = jnp.dot(a, b, preferred_element_type=jnp.float32)

        def rdma(src, dst, ssem, rsem, tgt):
            cp = pltpu.make_async_remote_copy(
                src_ref=src, dst_ref=dst, send_sem=ssem, recv_sem=rsem,
                device_id=(tgt,), device_id_type=pl.DeviceIdType.MESH,
            )
            cp.start()
            return cp

        for s in range(PLANE - 1):
            send_r = lax.rem(q - s + PLANE, PLANE)
            recv_r = lax.rem(q - s - 1 + PLANE, PLANE)
            send_l = lax.rem(q + s, PLANE)
            recv_l = lax.rem(q + s + 1, PLANE)
            p1_stage_r[s] = acc_ref[
                pl.ds(send_r * qr_rows, qr_rows), pl.ds(0, hn)
            ].astype(jnp.bfloat16)
            p1_stage_l[s] = acc_ref[
                pl.ds(send_l * qr_rows, qr_rows), pl.ds(hn, hn)
            ].astype(jnp.bfloat16)
            cp_r = rdma(p1_stage_r.at[s], p1_buf_r.at[s],
                        p1_send_r.at[s], p1_recv_r.at[s], qr)
            cp_l = rdma(p1_stage_l.at[s], p1_buf_l.at[s],
                        p1_send_l.at[s], p1_recv_l.at[s], ql)
            cp_r.wait()
            cp_l.wait()
            acc_ref[pl.ds(recv_r * qr_rows, qr_rows), pl.ds(0, hn)] = (
                acc_ref[pl.ds(recv_r * qr_rows, qr_rows), pl.ds(0, hn)]
                + p1_buf_r[s].astype(jnp.float32)
            )
            acc_ref[pl.ds(recv_l * qr_rows, qr_rows), pl.ds(hn, hn)] = (
                acc_ref[pl.ds(recv_l * qr_rows, qr_rows), pl.ds(hn, hn)]
                + p1_buf_l[s].astype(jnp.float32)
            )

        own_r = lax.rem(q + 1, PLANE)
        own_l = lax.rem(q + PLANE - 1, PLANE)
        base_r = own_r * qr_rows
        base_l = own_l * qr_rows
        sides = ((0, base_r), (1, base_l))
        col_off = (0, hn)

        b1 = jnp.bitwise_and(z, 1)
        b2 = jnp.bitwise_and(lax.div(z, 2), 1)
        keep1 = b1 * h2
        send1 = (1 - b1) * h2
        keep2 = keep1 + b2 * h4
        send2 = keep1 + (1 - b2) * h4

        cps = []
        for i, rbase in sides:
            sa_stage[i] = acc_ref[
                pl.ds(rbase + send1, h2), pl.ds(col_off[i], hn)
            ].astype(jnp.bfloat16)
            cps.append(rdma(sa_stage.at[i], sa_recv.at[i],
                            p2_send.at[i], p2_recv.at[i], pz1))
        for cp in cps:
            cp.wait()
        for i, rbase in sides:
            acc_ref[pl.ds(rbase + keep1, h2), pl.ds(col_off[i], hn)] = (
                acc_ref[pl.ds(rbase + keep1, h2), pl.ds(col_off[i], hn)]
                + sa_recv[i].astype(jnp.float32)
            )

        cps = []
        for i, rbase in sides:
            sb_stage[i] = acc_ref[
                pl.ds(rbase + send2, h4), pl.ds(col_off[i], hn)
            ].astype(jnp.bfloat16)
            cps.append(rdma(sb_stage.at[i], sb_recv.at[i],
                            p2_send.at[2 + i], p2_recv.at[2 + i], pz2))
        for cp in cps:
            cp.wait()
        for i, rbase in sides:
            acc_ref[pl.ds(rbase + keep2, h4), pl.ds(col_off[i], hn)] = (
                acc_ref[pl.ds(rbase + keep2, h4), pl.ds(col_off[i], hn)]
                + sb_recv[i].astype(jnp.float32)
            )

        pbs = (pb_r, pb_l)
        for i, rbase in sides:
            pbs[i][pl.ds(keep2, h4), :] = jnp.maximum(
                acc_ref[pl.ds(rbase + keep2, h4), pl.ds(col_off[i], hn)], 0.0
            ).astype(jnp.bfloat16)

        cps = []
        for i, _ in sides:
            cps.append(rdma(pbs[i].at[pl.ds(keep2, h4)],
                            pbs[i].at[pl.ds(keep2, h4)],
                            p2_send.at[4 + i], p2_recv.at[4 + i], pz2))
        for cp in cps:
            cp.wait()

        cps = []
        for i, _ in sides:
            cps.append(rdma(pbs[i].at[pl.ds(keep1, h2)],
                            pbs[i].at[pl.ds(keep1, h2)],
                            p2_send.at[6 + i], p2_recv.at[6 + i], pz1))
        for cp in cps:
            cp.wait()

        out_ref[pl.ds(base_r, qr_rows), pl.ds(0, hn)] = pb_r[...]
        out_ref[pl.ds(base_l, qr_rows), pl.ds(hn, hn)] = pb_l[...]

        for s in range(PLANE - 1):
            got_r = lax.rem(q - s + PLANE, PLANE)
            got_l = lax.rem(q + s, PLANE)
            cp_r = rdma(pb_r if s == 0 else agb_r.at[s - 1], agb_r.at[s],
                        p3_send_r.at[s], p3_recv_r.at[s], qr)
            cp_l = rdma(pb_l if s == 0 else agb_l.at[s - 1], agb_l.at[s],
                        p3_send_l.at[s], p3_recv_l.at[s], ql)
            cp_r.wait()
            cp_l.wait()
            out_ref[pl.ds(got_r * qr_rows, qr_rows), pl.ds(0, hn)] = agb_r[s]
            out_ref[pl.ds(got_l * qr_rows, qr_rows), pl.ds(hn, hn)] = agb_l[s]

        @functools.partial(pl.run_scoped, exit_sem=pltpu.SemaphoreType.REGULAR)
        def _(exit_sem):
            for nbr in neighbors:
                pl.semaphore_signal(exit_sem, inc=1, device_id=(nbr,),
                                    device_id_type=pl.DeviceIdType.MESH)
            pl.semaphore_wait(exit_sem, len(neighbors))

    return pl.pallas_call(
        body,
        out_shape=jax.ShapeDtypeStruct((m, n), jnp.bfloat16),
        in_specs=[
            pl.BlockSpec(memory_space=pltpu.VMEM),
            pl.BlockSpec(memory_space=pltpu.VMEM),
        ],
        out_specs=pl.BlockSpec(memory_space=pltpu.VMEM),
        scratch_shapes=[
            pltpu.VMEM((m, n), jnp.float32),
            pltpu.VMEM((PLANE - 1, qr_rows, hn), jnp.bfloat16),
            pltpu.VMEM((PLANE - 1, qr_rows, hn), jnp.bfloat16),
            pltpu.VMEM((PLANE - 1, qr_rows, hn), jnp.bfloat16),
            pltpu.VMEM((PLANE - 1, qr_rows, hn), jnp.bfloat16),
            pltpu.VMEM((2, h2, hn), jnp.bfloat16),
            pltpu.VMEM((2, h2, hn), jnp.bfloat16),
            pltpu.VMEM((2, h4, hn), jnp.bfloat16),
            pltpu.VMEM((2, h4, hn), jnp.bfloat16),
            pltpu.VMEM((qr_rows, hn), jnp.bfloat16),
            pltpu.VMEM((qr_rows, hn), jnp.bfloat16),
            pltpu.VMEM((PLANE - 1, qr_rows, hn), jnp.bfloat16),
            pltpu.VMEM((PLANE - 1, qr_rows, hn), jnp.bfloat16),
            pltpu.SemaphoreType.DMA((PLANE - 1,)),
            pltpu.SemaphoreType.DMA((PLANE - 1,)),
            pltpu.SemaphoreType.DMA((PLANE - 1,)),
            pltpu.SemaphoreType.DMA((PLANE - 1,)),
            pltpu.SemaphoreType.DMA((8,)),
            pltpu.SemaphoreType.DMA((8,)),
            pltpu.SemaphoreType.DMA((PLANE - 1,)),
            pltpu.SemaphoreType.DMA((PLANE - 1,)),
            pltpu.SemaphoreType.DMA((PLANE - 1,)),
            pltpu.SemaphoreType.DMA((PLANE - 1,)),
        ],
        compiler_params=pltpu.CompilerParams(collective_id=0),
    )(A, B)


# device time: 97022 ns/iter; 2.6938x vs baseline; 1.0131x over previous
import functools

import jax
import jax.numpy as jnp
from jax import lax
from jax.experimental import pallas as pl
from jax.experimental.pallas import tpu as pltpu

N_DEV = 16
PLANE = 4
NZ = 4


def kernel(A, B):
    m, k = A.shape
    _, n = B.shape
    hn = n // 2
    qr_rows = m // PLANE
    h2 = qr_rows // 2
    h4 = qr_rows // 4

    def body(a_ref, b_ref, out_ref, acc_r, acc_l,
             p1_buf_r, p1_buf_l, sa_recv, sb_recv, agb_r, agb_l,
             p1_send_r, p1_recv_r, p1_send_l, p1_recv_l,
             p2_send, p2_recv,
             p3_send_r, p3_recv_r, p3_send_l, p3_recv_l):
        my = lax.axis_index("i")
        z = lax.div(my, PLANE)
        q = lax.rem(my, PLANE)
        qr = z * PLANE + lax.rem(q + 1, PLANE)
        ql = z * PLANE + lax.rem(q + PLANE - 1, PLANE)
        pz1 = q + PLANE * jnp.bitwise_xor(z, 1)
        pz2 = q + PLANE * jnp.bitwise_xor(z, 2)
        neighbors = (qr, ql, pz1, pz2)

        barrier = pltpu.get_barrier_semaphore()
        for nbr in neighbors:
            pl.semaphore_signal(barrier, inc=1, device_id=(nbr,),
                                device_id_type=pl.DeviceIdType.MESH)
        pl.semaphore_wait(barrier, len(neighbors))

        a = a_ref[...].astype(jnp.bfloat16)
        b = b_ref[...].astype(jnp.bfloat16)
        acc_r[...] = jnp.dot(
            a, b[:, :hn], preferred_element_type=jnp.float32
        ).astype(jnp.bfloat16)
        acc_l[...] = jnp.dot(
            a, b[:, hn:], preferred_element_type=jnp.float32
        ).astype(jnp.bfloat16)

        def rdma(src, dst, ssem, rsem, tgt):
            cp = pltpu.make_async_remote_copy(
                src_ref=src, dst_ref=dst, send_sem=ssem, recv_sem=rsem,
                device_id=(tgt,), device_id_type=pl.DeviceIdType.MESH,
            )
            cp.start()
            return cp

        for s in range(PLANE - 1):
            send_r = lax.rem(q - s + PLANE, PLANE)
            recv_r = lax.rem(q - s - 1 + PLANE, PLANE)
            send_l = lax.rem(q + s, PLANE)
            recv_l = lax.rem(q + s + 1, PLANE)
            cp_r = rdma(acc_r.at[pl.ds(send_r * qr_rows, qr_rows)],
                        p1_buf_r.at[s],
                        p1_send_r.at[s], p1_recv_r.at[s], qr)
            cp_l = rdma(acc_l.at[pl.ds(send_l * qr_rows, qr_rows)],
                        p1_buf_l.at[s],
                        p1_send_l.at[s], p1_recv_l.at[s], ql)
            cp_r.wait()
            cp_l.wait()
            acc_r[pl.ds(recv_r * qr_rows, qr_rows), :] = (
                acc_r[pl.ds(recv_r * qr_rows, qr_rows), :] + p1_buf_r[s]
            )
            acc_l[pl.ds(recv_l * qr_rows, qr_rows), :] = (
                acc_l[pl.ds(recv_l * qr_rows, qr_rows), :] + p1_buf_l[s]
            )

        base_r = lax.rem(q + 1, PLANE) * qr_rows
        base_l = lax.rem(q + PLANE - 1, PLANE) * qr_rows
        accs = (acc_r, acc_l)
        bases = (base_r, base_l)

        b1 = jnp.bitwise_and(z, 1)
        b2 = jnp.bitwise_and(lax.div(z, 2), 1)
        keep1 = b1 * h2
        send1 = (1 - b1) * h2
        keep2 = keep1 + b2 * h4
        send2 = keep1 + (1 - b2) * h4

        cps = [rdma(accs[i].at[pl.ds(bases[i] + send1, h2)], sa_recv.at[i],
                    p2_send.at[i], p2_recv.at[i], pz1) for i in range(2)]
        for cp in cps:
            cp.wait()
        for i in range(2):
            accs[i][pl.ds(bases[i] + keep1, h2), :] = (
                accs[i][pl.ds(bases[i] + keep1, h2), :] + sa_recv[i]
            )

        cps = [rdma(accs[i].at[pl.ds(bases[i] + send2, h4)], sb_recv.at[i],
                    p2_send.at[2 + i], p2_recv.at[2 + i], pz2)
               for i in range(2)]
        for cp in cps:
            cp.wait()
        for i in range(2):
            accs[i][pl.ds(bases[i] + keep2, h4), :] = (
                accs[i][pl.ds(bases[i] + keep2, h4), :] + sb_recv[i]
            )

        for i in range(2):
            accs[i][pl.ds(bases[i] + keep2, h4), :] = jnp.maximum(
                accs[i][pl.ds(bases[i] + keep2, h4), :], 0.0
            )

        cps = [rdma(accs[i].at[pl.ds(bases[i] + keep2, h4)],
                    accs[i].at[pl.ds(bases[i] + keep2, h4)],
                    p2_send.at[4 + i], p2_recv.at[4 + i], pz2)
               for i in range(2)]
        for cp in cps:
            cp.wait()

        cps = [rdma(accs[i].at[pl.ds(bases[i] + keep1, h2)],
                    accs[i].at[pl.ds(bases[i] + keep1, h2)],
                    p2_send.at[6 + i], p2_recv.at[6 + i], pz1)
               for i in range(2)]
        for cp in cps:
            cp.wait()

        out_ref[pl.ds(base_r, qr_rows), pl.ds(0, hn)] = acc_r[
            pl.ds(base_r, qr_rows), :
        ]
        out_ref[pl.ds(base_l, qr_rows), pl.ds(hn, hn)] = acc_l[
            pl.ds(base_l, qr_rows), :
        ]

        for s in range(PLANE - 1):
            got_r = lax.rem(q - s + PLANE, PLANE)
            got_l = lax.rem(q + s, PLANE)
            cp_r = rdma(acc_r.at[pl.ds(base_r, qr_rows)] if s == 0
                        else agb_r.at[s - 1],
                        agb_r.at[s], p3_send_r.at[s], p3_recv_r.at[s], qr)
            cp_l = rdma(acc_l.at[pl.ds(base_l, qr_rows)] if s == 0
                        else agb_l.at[s - 1],
                        agb_l.at[s], p3_send_l.at[s], p3_recv_l.at[s], ql)
            cp_r.wait()
            cp_l.wait()
            out_ref[pl.ds(got_r * qr_rows, qr_rows), pl.ds(0, hn)] = agb_r[s]
            out_ref[pl.ds(got_l * qr_rows, qr_rows), pl.ds(hn, hn)] = agb_l[s]

        @functools.partial(pl.run_scoped, exit_sem=pltpu.SemaphoreType.REGULAR)
        def _(exit_sem):
            for nbr in neighbors:
                pl.semaphore_signal(exit_sem, inc=1, device_id=(nbr,),
                                    device_id_type=pl.DeviceIdType.MESH)
            pl.semaphore_wait(exit_sem, len(neighbors))

    return pl.pallas_call(
        body,
        out_shape=jax.ShapeDtypeStruct((m, n), jnp.bfloat16),
        in_specs=[
            pl.BlockSpec(memory_space=pltpu.VMEM),
            pl.BlockSpec(memory_space=pltpu.VMEM),
        ],
        out_specs=pl.BlockSpec(memory_space=pltpu.VMEM),
        scratch_shapes=[
            pltpu.VMEM((m, hn), jnp.bfloat16),
            pltpu.VMEM((m, hn), jnp.bfloat16),
            pltpu.VMEM((PLANE - 1, qr_rows, hn), jnp.bfloat16),
            pltpu.VMEM((PLANE - 1, qr_rows, hn), jnp.bfloat16),
            pltpu.VMEM((2, h2, hn), jnp.bfloat16),
            pltpu.VMEM((2, h4, hn), jnp.bfloat16),
            pltpu.VMEM((PLANE - 1, qr_rows, hn), jnp.bfloat16),
            pltpu.VMEM((PLANE - 1, qr_rows, hn), jnp.bfloat16),
            pltpu.SemaphoreType.DMA((PLANE - 1,)),
            pltpu.SemaphoreType.DMA((PLANE - 1,)),
            pltpu.SemaphoreType.DMA((PLANE - 1,)),
            pltpu.SemaphoreType.DMA((PLANE - 1,)),
            pltpu.SemaphoreType.DMA((8,)),
            pltpu.SemaphoreType.DMA((8,)),
            pltpu.SemaphoreType.DMA((PLANE - 1,)),
            pltpu.SemaphoreType.DMA((PLANE - 1,)),
            pltpu.SemaphoreType.DMA((PLANE - 1,)),
            pltpu.SemaphoreType.DMA((PLANE - 1,)),
        ],
        compiler_params=pltpu.CompilerParams(collective_id=0),
    )(A, B)


# device time: 93757 ns/iter; 2.7876x vs baseline; 1.0348x over previous
import functools

import jax
import jax.numpy as jnp
from jax import lax
from jax.experimental import pallas as pl
from jax.experimental.pallas import tpu as pltpu

N_DEV = 16
PLANE = 4
NZ = 4


def kernel(A, B):
    m, k = A.shape
    _, n = B.shape
    hn = n // 2
    qr_rows = m // PLANE
    h2 = qr_rows // 2
    h4 = qr_rows // 4

    def body(a_ref, b_ref, out_ref, acc_r, acc_l,
             p1_buf_r, p1_buf_l, sa_recv, sb_recv,
             p1_send_r, p1_recv_r, p1_send_l, p1_recv_l,
             p2_send, p2_recv,
             p3_send_r, p3_recv_r, p3_send_l, p3_recv_l):
        my = lax.axis_index("i")
        z = lax.div(my, PLANE)
        q = lax.rem(my, PLANE)
        qr = z * PLANE + lax.rem(q + 1, PLANE)
        ql = z * PLANE + lax.rem(q + PLANE - 1, PLANE)
        pz1 = q + PLANE * jnp.bitwise_xor(z, 1)
        pz2 = q + PLANE * jnp.bitwise_xor(z, 2)
        neighbors = (qr, ql, pz1, pz2)

        barrier = pltpu.get_barrier_semaphore()
        for nbr in neighbors:
            pl.semaphore_signal(barrier, inc=1, device_id=(nbr,),
                                device_id_type=pl.DeviceIdType.MESH)
        pl.semaphore_wait(barrier, len(neighbors))

        b_bf = b_ref[...].astype(jnp.bfloat16)

        def compute_quarter(acc, idx, cols):
            a_q = a_ref[pl.ds(idx * qr_rows, qr_rows), :].astype(jnp.bfloat16)
            acc[pl.ds(idx * qr_rows, qr_rows), :] = jnp.dot(
                a_q, b_bf[:, cols], preferred_element_type=jnp.float32
            ).astype(jnp.bfloat16)

        def rdma(src, dst, ssem, rsem, tgt):
            cp = pltpu.make_async_remote_copy(
                src_ref=src, dst_ref=dst, send_sem=ssem, recv_sem=rsem,
                device_id=(tgt,), device_id_type=pl.DeviceIdType.MESH,
            )
            cp.start()
            return cp

        compute_quarter(acc_r, q, slice(0, hn))
        compute_quarter(acc_l, q, slice(hn, n))
        for s in range(PLANE - 1):
            send_r = lax.rem(q - s + PLANE, PLANE)
            recv_r = lax.rem(q - s - 1 + PLANE, PLANE)
            send_l = lax.rem(q + s, PLANE)
            recv_l = lax.rem(q + s + 1, PLANE)
            cp_r = rdma(acc_r.at[pl.ds(send_r * qr_rows, qr_rows)],
                        p1_buf_r.at[s],
                        p1_send_r.at[s], p1_recv_r.at[s], qr)
            cp_l = rdma(acc_l.at[pl.ds(send_l * qr_rows, qr_rows)],
                        p1_buf_l.at[s],
                        p1_send_l.at[s], p1_recv_l.at[s], ql)
            compute_quarter(acc_r, recv_r, slice(0, hn))
            compute_quarter(acc_l, recv_l, slice(hn, n))
            cp_r.wait()
            acc_r[pl.ds(recv_r * qr_rows, qr_rows), :] = (
                acc_r[pl.ds(recv_r * qr_rows, qr_rows), :] + p1_buf_r[s]
            )
            cp_l.wait()
            acc_l[pl.ds(recv_l * qr_rows, qr_rows), :] = (
                acc_l[pl.ds(recv_l * qr_rows, qr_rows), :] + p1_buf_l[s]
            )

        base_r = lax.rem(q + 1, PLANE) * qr_rows
        base_l = lax.rem(q + PLANE - 1, PLANE) * qr_rows
        accs = (acc_r, acc_l)
        bases = (base_r, base_l)

        b1 = jnp.bitwise_and(z, 1)
        b2 = jnp.bitwise_and(lax.div(z, 2), 1)
        keep1 = b1 * h2
        send1 = (1 - b1) * h2
        keep2 = keep1 + b2 * h4
        send2 = keep1 + (1 - b2) * h4

        cps = [rdma(accs[i].at[pl.ds(bases[i] + send1, h2)], sa_recv.at[i],
                    p2_send.at[i], p2_recv.at[i], pz1) for i in range(2)]
        for i in range(2):
            cps[i].wait()
            accs[i][pl.ds(bases[i] + keep1, h2), :] = (
                accs[i][pl.ds(bases[i] + keep1, h2), :] + sa_recv[i]
            )

        cps = [rdma(accs[i].at[pl.ds(bases[i] + send2, h4)], sb_recv.at[i],
                    p2_send.at[2 + i], p2_recv.at[2 + i], pz2)
               for i in range(2)]
        for i in range(2):
            cps[i].wait()
            accs[i][pl.ds(bases[i] + keep2, h4), :] = jnp.maximum(
                accs[i][pl.ds(bases[i] + keep2, h4), :] + sb_recv[i], 0.0
            )

        cps = [rdma(accs[i].at[pl.ds(bases[i] + keep2, h4)],
                    accs[i].at[pl.ds(bases[i] + keep2, h4)],
                    p2_send.at[4 + i], p2_recv.at[4 + i], pz2)
               for i in range(2)]
        for cp in cps:
            cp.wait()

        cps = [rdma(accs[i].at[pl.ds(bases[i] + keep1, h2)],
                    accs[i].at[pl.ds(bases[i] + keep1, h2)],
                    p2_send.at[6 + i], p2_recv.at[6 + i], pz1)
               for i in range(2)]
        for cp in cps:
            cp.wait()

        col_r = pl.ds(0, hn)
        col_l = pl.ds(hn, hn)
        cp_r = rdma(acc_r.at[pl.ds(base_r, qr_rows)],
                    out_ref.at[pl.ds(base_r, qr_rows), col_r],
                    p3_send_r.at[0], p3_recv_r.at[0], qr)
        cp_l = rdma(acc_l.at[pl.ds(base_l, qr_rows)],
                    out_ref.at[pl.ds(base_l, qr_rows), col_l],
                    p3_send_l.at[0], p3_recv_l.at[0], ql)
        out_ref[pl.ds(base_r, qr_rows), col_r] = acc_r[pl.ds(base_r, qr_rows), :]
        out_ref[pl.ds(base_l, qr_rows), col_l] = acc_l[pl.ds(base_l, qr_rows), :]
        for s in range(PLANE - 1):
            got_r = lax.rem(q - s + PLANE, PLANE) * qr_rows
            got_l = lax.rem(q + s, PLANE) * qr_rows
            cp_r.wait()
            cp_l.wait()
            if s < PLANE - 2:
                cp_r = rdma(out_ref.at[pl.ds(got_r, qr_rows), col_r],
                            out_ref.at[pl.ds(got_r, qr_rows), col_r],
                            p3_send_r.at[s + 1], p3_recv_r.at[s + 1], qr)
                cp_l = rdma(out_ref.at[pl.ds(got_l, qr_rows), col_l],
                            out_ref.at[pl.ds(got_l, qr_rows), col_l],
                            p3_send_l.at[s + 1], p3_recv_l.at[s + 1], ql)

        @functools.partial(pl.run_scoped, exit_sem=pltpu.SemaphoreType.REGULAR)
        def _(exit_sem):
            for nbr in neighbors:
                pl.semaphore_signal(exit_sem, inc=1, device_id=(nbr,),
                                    device_id_type=pl.DeviceIdType.MESH)
            pl.semaphore_wait(exit_sem, len(neighbors))

    return pl.pallas_call(
        body,
        out_shape=jax.ShapeDtypeStruct((m, n), jnp.bfloat16),
        in_specs=[
            pl.BlockSpec(memory_space=pltpu.VMEM),
            pl.BlockSpec(memory_space=pltpu.VMEM),
        ],
        out_specs=pl.BlockSpec(memory_space=pltpu.VMEM),
        scratch_shapes=[
            pltpu.VMEM((m, hn), jnp.bfloat16),
            pltpu.VMEM((m, hn), jnp.bfloat16),
            pltpu.VMEM((PLANE - 1, qr_rows, hn), jnp.bfloat16),
            pltpu.VMEM((PLANE - 1, qr_rows, hn), jnp.bfloat16),
            pltpu.VMEM((2, h2, hn), jnp.bfloat16),
            pltpu.VMEM((2, h4, hn), jnp.bfloat16),
            pltpu.SemaphoreType.DMA((PLANE - 1,)),
            pltpu.SemaphoreType.DMA((PLANE - 1,)),
            pltpu.SemaphoreType.DMA((PLANE - 1,)),
            pltpu.SemaphoreType.DMA((PLANE - 1,)),
            pltpu.SemaphoreType.DMA((8,)),
            pltpu.SemaphoreType.DMA((8,)),
            pltpu.SemaphoreType.DMA((PLANE - 1,)),
            pltpu.SemaphoreType.DMA((PLANE - 1,)),
            pltpu.SemaphoreType.DMA((PLANE - 1,)),
            pltpu.SemaphoreType.DMA((PLANE - 1,)),
        ],
        compiler_params=pltpu.CompilerParams(collective_id=0),
    )(A, B)


# device time: 85901 ns/iter; 3.0426x vs baseline; 1.0915x over previous
import functools

import jax
import jax.numpy as jnp
from jax import lax
from jax.experimental import pallas as pl
from jax.experimental.pallas import tpu as pltpu

N_DEV = 16
PLANE = 4
NZ = 4


def kernel(A, B):
    m, k = A.shape
    _, n = B.shape
    hn = n // 2
    qr_rows = m // PLANE
    h2 = qr_rows // 2
    h4 = qr_rows // 4

    def body(a_ref, b_ref, out_ref, acc_r, acc_l,
             p1_buf_r, p1_buf_l, sa_recv, sb_recv,
             p1_send_r, p1_recv_r, p1_send_l, p1_recv_l,
             p2_send, p2_recv,
             p3_send_r, p3_recv_r, p3_send_l, p3_recv_l):
        my = lax.axis_index("i")
        z = lax.div(my, PLANE)
        q = lax.rem(my, PLANE)
        qr = z * PLANE + lax.rem(q + 1, PLANE)
        ql = z * PLANE + lax.rem(q + PLANE - 1, PLANE)
        pz1 = q + PLANE * jnp.bitwise_xor(z, 1)
        pz2 = q + PLANE * jnp.bitwise_xor(z, 2)
        neighbors = (qr, ql, pz1, pz2)

        barrier = pltpu.get_barrier_semaphore()
        for nbr in neighbors:
            pl.semaphore_signal(barrier, inc=1, device_id=(nbr,),
                                device_id_type=pl.DeviceIdType.MESH)
        pl.semaphore_wait(barrier, len(neighbors))

        b_bf = b_ref[...].astype(jnp.bfloat16)

        def compute_quarter(acc, idx, cols):
            a_q = a_ref[pl.ds(idx * qr_rows, qr_rows), :].astype(jnp.bfloat16)
            acc[pl.ds(idx * qr_rows, qr_rows), :] = jnp.dot(
                a_q, b_bf[:, cols], preferred_element_type=jnp.float32
            ).astype(jnp.bfloat16)

        def rdma(src, dst, ssem, rsem, tgt):
            cp = pltpu.make_async_remote_copy(
                src_ref=src, dst_ref=dst, send_sem=ssem, recv_sem=rsem,
                device_id=(tgt,), device_id_type=pl.DeviceIdType.MESH,
            )
            cp.start()
            return cp

        compute_quarter(acc_r, q, slice(0, hn))
        compute_quarter(acc_l, q, slice(hn, n))
        for s in range(PLANE - 1):
            send_r = lax.rem(q - s + PLANE, PLANE)
            recv_r = lax.rem(q - s - 1 + PLANE, PLANE)
            send_l = lax.rem(q + s, PLANE)
            recv_l = lax.rem(q + s + 1, PLANE)
            cp_r = rdma(acc_r.at[pl.ds(send_r * qr_rows, qr_rows)],
                        p1_buf_r.at[s],
                        p1_send_r.at[s], p1_recv_r.at[s], qr)
            cp_l = rdma(acc_l.at[pl.ds(send_l * qr_rows, qr_rows)],
                        p1_buf_l.at[s],
                        p1_send_l.at[s], p1_recv_l.at[s], ql)
            compute_quarter(acc_r, recv_r, slice(0, hn))
            compute_quarter(acc_l, recv_l, slice(hn, n))
            cp_r.wait()
            acc_r[pl.ds(recv_r * qr_rows, qr_rows), :] = (
                acc_r[pl.ds(recv_r * qr_rows, qr_rows), :] + p1_buf_r[s]
            )
            cp_l.wait()
            acc_l[pl.ds(recv_l * qr_rows, qr_rows), :] = (
                acc_l[pl.ds(recv_l * qr_rows, qr_rows), :] + p1_buf_l[s]
            )

        base_r = lax.rem(q + 1, PLANE) * qr_rows
        base_l = lax.rem(q + PLANE - 1, PLANE) * qr_rows
        accs = (acc_r, acc_l)
        bases = (base_r, base_l)

        b1 = jnp.bitwise_and(z, 1)
        b2 = jnp.bitwise_and(lax.div(z, 2), 1)
        keep1 = b1 * h2
        send1 = (1 - b1) * h2
        keep2 = keep1 + b2 * h4
        send2 = keep1 + (1 - b2) * h4

        cps = [rdma(accs[i].at[pl.ds(bases[i] + send1, h2)], sa_recv.at[i],
                    p2_send.at[i], p2_recv.at[i], pz1) for i in range(2)]
        for i in range(2):
            cps[i].wait()
            accs[i][pl.ds(bases[i] + keep1, h2), :] = (
                accs[i][pl.ds(bases[i] + keep1, h2), :] + sa_recv[i]
            )

        cps = [rdma(accs[i].at[pl.ds(bases[i] + send2, h4)], sb_recv.at[i],
                    p2_send.at[2 + i], p2_recv.at[2 + i], pz2)
               for i in range(2)]
        for i in range(2):
            cps[i].wait()
            accs[i][pl.ds(bases[i] + keep2, h4), :] = jnp.maximum(
                accs[i][pl.ds(bases[i] + keep2, h4), :] + sb_recv[i], 0.0
            )

        col_ = (pl.ds(0, hn), pl.ds(hn, hn))
        sends = (p3_send_r, p3_send_l)
        recvs = (p3_recv_r, p3_recv_l)
        tgt_ = (qr, ql)
        pieces = ((0, keep2, h4), (1, send2, h4), (2, send1, h2))

        def ag_hop(i, chain, off, nrows, s):
            if s == 0:
                src = accs[i].at[pl.ds(bases[i] + off, nrows)]
                dst_rows = pl.ds(bases[i] + off, nrows)
            else:
                back = lax.rem(q - s + 1 + PLANE, PLANE) if i == 0 else \
                    lax.rem(q + s - 1 + PLANE, PLANE)
                rows = pl.ds(back * qr_rows + off, nrows)
                src = out_ref.at[rows, col_[i]]
                dst_rows = rows
            return rdma(src, out_ref.at[dst_rows, col_[i]],
                        sends[i].at[3 * chain + s],
                        recvs[i].at[3 * chain + s], tgt_[i])

        cp_c = [rdma(accs[i].at[pl.ds(bases[i] + keep2, h4)],
                     accs[i].at[pl.ds(bases[i] + keep2, h4)],
                     p2_send.at[4 + i], p2_recv.at[4 + i], pz2)
                for i in range(2)]
        p0 = [ag_hop(i, 0, keep2, h4, 0) for i in range(2)]
        for i in range(2):
            out_ref[pl.ds(bases[i] + keep2, h4), col_[i]] = accs[i][
                pl.ds(bases[i] + keep2, h4), :
            ]
        for cp in cp_c:
            cp.wait()
        cp_d = [rdma(accs[i].at[pl.ds(bases[i] + keep1, h2)],
                     accs[i].at[pl.ds(bases[i] + keep1, h2)],
                     p2_send.at[6 + i], p2_recv.at[6 + i], pz1)
                for i in range(2)]
        p1 = [ag_hop(i, 1, send2, h4, 0) for i in range(2)]
        for i in range(2):
            out_ref[pl.ds(bases[i] + send2, h4), col_[i]] = accs[i][
                pl.ds(bases[i] + send2, h4), :
            ]
        for i in range(2):
            p0[i].wait()
        p0 = [ag_hop(i, 0, keep2, h4, 1) for i in range(2)]
        for cp in cp_d:
            cp.wait()
        p2 = [ag_hop(i, 2, send1, h2, 0) for i in range(2)]
        for i in range(2):
            out_ref[pl.ds(bases[i] + send1, h2), col_[i]] = accs[i][
                pl.ds(bases[i] + send1, h2), :
            ]
        chains = {0: p0, 1: p1, 2: p2}
        hop_at = {0: 1, 1: 0, 2: 0}
        for chain, off, nrows in ((1, send2, h4), (0, keep2, h4),
                                  (2, send1, h2), (1, send2, h4),
                                  (0, keep2, h4), (2, send1, h2),
                                  (1, send2, h4), (2, send1, h2)):
            for i in range(2):
                chains[chain][i].wait()
            s = hop_at[chain] + 1
            hop_at[chain] = s
            if s <= PLANE - 2:
                chains[chain] = [ag_hop(i, chain, off, nrows, s)
                                 for i in range(2)]

        @functools.partial(pl.run_scoped, exit_sem=pltpu.SemaphoreType.REGULAR)
        def _(exit_sem):
            for nbr in neighbors:
                pl.semaphore_signal(exit_sem, inc=1, device_id=(nbr,),
                                    device_id_type=pl.DeviceIdType.MESH)
            pl.semaphore_wait(exit_sem, len(neighbors))

    return pl.pallas_call(
        body,
        out_shape=jax.ShapeDtypeStruct((m, n), jnp.bfloat16),
        in_specs=[
            pl.BlockSpec(memory_space=pltpu.VMEM),
            pl.BlockSpec(memory_space=pltpu.VMEM),
        ],
        out_specs=pl.BlockSpec(memory_space=pltpu.VMEM),
        scratch_shapes=[
            pltpu.VMEM((m, hn), jnp.bfloat16),
            pltpu.VMEM((m, hn), jnp.bfloat16),
            pltpu.VMEM((PLANE - 1, qr_rows, hn), jnp.bfloat16),
            pltpu.VMEM((PLANE - 1, qr_rows, hn), jnp.bfloat16),
            pltpu.VMEM((2, h2, hn), jnp.bfloat16),
            pltpu.VMEM((2, h4, hn), jnp.bfloat16),
            pltpu.SemaphoreType.DMA((PLANE - 1,)),
            pltpu.SemaphoreType.DMA((PLANE - 1,)),
            pltpu.SemaphoreType.DMA((PLANE - 1,)),
            pltpu.SemaphoreType.DMA((PLANE - 1,)),
            pltpu.SemaphoreType.DMA((8,)),
            pltpu.SemaphoreType.DMA((8,)),
            pltpu.SemaphoreType.DMA((9,)),
            pltpu.SemaphoreType.DMA((9,)),
            pltpu.SemaphoreType.DMA((9,)),
            pltpu.SemaphoreType.DMA((9,)),
        ],
        compiler_params=pltpu.CompilerParams(collective_id=0),
    )(A, B)


# device time: 72045 ns/iter; 3.6277x vs baseline; 1.1923x over previous
import functools

import jax
import jax.numpy as jnp
from jax import lax
from jax.experimental import pallas as pl
from jax.experimental.pallas import tpu as pltpu

N_DEV = 16
PLANE = 4
NZ = 4


def kernel(A, B):
    m, k = A.shape
    _, n = B.shape
    hn = n // 2
    qr_rows = m // PLANE
    h2 = qr_rows // 2
    h4 = qr_rows // 4

    def body(a_ref, b_ref, out_ref, acc_r, acc_l,
             px_buf_r, px_buf_l, py_buf_r, py_buf_l, sa_recv, sb_recv,
             p1_send_r, p1_recv_r, p1_send_l, p1_recv_l,
             p2_send, p2_recv,
             p3_send_r, p3_recv_r, p3_send_l, p3_recv_l):
        my = lax.axis_index("i")
        z = lax.div(my, PLANE)
        q = lax.rem(my, PLANE)
        qr = z * PLANE + lax.rem(q + 1, PLANE)
        ql = z * PLANE + lax.rem(q + PLANE - 1, PLANE)
        pz1 = q + PLANE * jnp.bitwise_xor(z, 1)
        pz2 = q + PLANE * jnp.bitwise_xor(z, 2)
        neighbors = (qr, ql, pz1, pz2)

        barrier = pltpu.get_barrier_semaphore()
        for nbr in neighbors:
            pl.semaphore_signal(barrier, inc=1, device_id=(nbr,),
                                device_id_type=pl.DeviceIdType.MESH)
        pl.semaphore_wait(barrier, len(neighbors))

        b_bf = b_ref[...].astype(jnp.bfloat16)

        def compute_quarter(acc, idx, cols):
            a_q = a_ref[pl.ds(idx * qr_rows, qr_rows), :].astype(jnp.bfloat16)
            acc[pl.ds(idx * qr_rows, qr_rows), :] = jnp.dot(
                a_q, b_bf[:, cols], preferred_element_type=jnp.float32
            ).astype(jnp.bfloat16)

        def rdma(src, dst, ssem, rsem, tgt):
            cp = pltpu.make_async_remote_copy(
                src_ref=src, dst_ref=dst, send_sem=ssem, recv_sem=rsem,
                device_id=(tgt,), device_id_type=pl.DeviceIdType.MESH,
            )
            cp.start()
            return cp

        b1 = jnp.bitwise_and(z, 1)
        b2 = jnp.bitwise_and(lax.div(z, 2), 1)
        keep1 = b1 * h2
        send1 = (1 - b1) * h2
        d_keep = b2 * h4
        d_send = (1 - b2) * h4
        keep2 = keep1 + d_keep
        send2 = keep1 + d_send

        accs = (acc_r, acc_l)
        xbufs = (px_buf_r, px_buf_l)
        ybufs = (py_buf_r, py_buf_l)
        p1s = (p1_send_r, p1_send_l)
        p1r = (p1_recv_r, p1_recv_l)
        tgt_ = (qr, ql)

        def chunk_q(i, s, delta):
            d = -s + delta if i == 0 else s + delta
            return lax.rem(q + d + 2 * PLANE, PLANE)

        def p1_hop(i, sub, s):
            off = send1 if sub == 0 else keep1
            buf = xbufs[i] if sub == 0 else ybufs[i]
            sem = 3 * sub + s
            src_q = chunk_q(i, s, 0)
            return rdma(
                accs[i].at[pl.ds(src_q * qr_rows + off, h2)], buf.at[s],
                p1s[i].at[sem], p1r[i].at[sem], tgt_[i])

        def p1_add(i, sub, s):
            off = send1 if sub == 0 else keep1
            buf = xbufs[i] if sub == 0 else ybufs[i]
            dst_q = chunk_q(i, s, -1 if i == 0 else 1)
            rows = pl.ds(dst_q * qr_rows + off, h2)
            accs[i][rows, :] = accs[i][rows, :] + buf[s]

        cols_ = (slice(0, hn), slice(hn, n))
        compute_quarter(acc_r, q, cols_[0])
        compute_quarter(acc_l, q, cols_[1])
        hops = {}
        for i in range(2):
            for sub in range(2):
                hops[(i, sub)] = p1_hop(i, sub, 0)

        base_r = lax.rem(q + 1, PLANE) * qr_rows
        base_l = lax.rem(q + PLANE - 1, PLANE) * qr_rows
        bases = (base_r, base_l)

        def z_reduce(i, step_idx, off, nrows, buf, tgt):
            return rdma(accs[i].at[pl.ds(bases[i] + off, nrows)],
                        buf,
                        p2_send.at[2 * step_idx + i],
                        p2_recv.at[2 * step_idx + i], tgt)

        def z_gather(i, step_idx, off, nrows, tgt):
            return rdma(accs[i].at[pl.ds(bases[i] + off, nrows)],
                        accs[i].at[pl.ds(bases[i] + off, nrows)],
                        p2_send.at[2 * step_idx + i],
                        p2_recv.at[2 * step_idx + i], tgt)

        za = {}
        for s in range(PLANE - 1):
            compute_quarter(acc_r, chunk_q(0, s, -1), cols_[0])
            compute_quarter(acc_l, chunk_q(1, s, 1), cols_[1])
            for i in range(2):
                hops[(i, 0)].wait()
                p1_add(i, 0, s)
                if s < PLANE - 2:
                    hops[(i, 0)] = p1_hop(i, 0, s + 1)
                else:
                    za[(i, 0)] = z_reduce(i, 0, send1 + d_send, h4,
                                          sa_recv.at[2 * i + 0], pz1)
                    za[(i, 1)] = z_reduce(i, 1, send1 + d_keep, h4,
                                          sa_recv.at[2 * i + 1], pz1)
            for i in range(2):
                hops[(i, 1)].wait()
                p1_add(i, 1, s)
                if s < PLANE - 2:
                    hops[(i, 1)] = p1_hop(i, 1, s + 1)

        zb = {}
        for i in range(2):
            za[(i, 0)].wait()
            rows = pl.ds(bases[i] + keep1 + d_send, h4)
            accs[i][rows, :] = accs[i][rows, :] + sa_recv[2 * i + 0]
        for i in range(2):
            zb[i] = z_reduce(i, 2, send2, h4, sb_recv.at[i], pz2)
            za[(i, 1)].wait()
            rows = pl.ds(bases[i] + keep1 + d_keep, h4)
            accs[i][rows, :] = accs[i][rows, :] + sa_recv[2 * i + 1]

        col_ = (pl.ds(0, hn), pl.ds(hn, hn))
        p3s = (p3_send_r, p3_send_l)
        p3r = (p3_recv_r, p3_recv_l)

        def ag_hop(i, chain, off, s):
            if s == 0:
                rows = pl.ds(bases[i] + off, h4)
                src = accs[i].at[rows]
            else:
                back = chunk_q(i, s - 1, 0)
                rows = pl.ds(back * qr_rows + off, h4)
                src = out_ref.at[rows, col_[i]]
            return rdma(src, out_ref.at[rows, col_[i]],
                        p3s[i].at[3 * chain + s],
                        p3r[i].at[3 * chain + s], tgt_[i])

        def out_own(i, off):
            rows = pl.ds(bases[i] + off, h4)
            out_ref[rows, col_[i]] = accs[i][rows, :]

        zc, zd1, zd2 = {}, {}, {}
        ag = {}
        for i in range(2):
            zb[i].wait()
            rows = pl.ds(bases[i] + keep2, h4)
            accs[i][rows, :] = jnp.maximum(
                accs[i][rows, :] + sb_recv[i], 0.0)
            zc[i] = z_gather(i, 3, keep2, h4, pz2)
            zd1[i] = z_gather(i, 4, keep2, h4, pz1)
            ag[(i, 0)] = ag_hop(i, 0, keep2, 0)
            out_own(i, keep2)

        for i in range(2):
            zc[i].wait()
            zd2[i] = z_gather(i, 5, send2, h4, pz1)
            ag[(i, 1)] = ag_hop(i, 1, send2, 0)
            out_own(i, send2)
        for i in range(2):
            ag[(i, 0)].wait()
            ag[(i, 0)] = ag_hop(i, 0, keep2, 1)
        for i in range(2):
            zd1[i].wait()
            ag[(i, 2)] = ag_hop(i, 2, send1 + d_keep, 0)
            out_own(i, send1 + d_keep)
        for i in range(2):
            zd2[i].wait()
            ag[(i, 3)] = ag_hop(i, 3, send1 + d_send, 0)
            out_own(i, send1 + d_send)

        offs = (keep2, send2, send1 + d_keep, send1 + d_send)
        hop_at = {0: 1, 1: 0, 2: 0, 3: 0}
        for chain in (1, 0, 2, 1, 3, 0, 2, 1, 3, 2, 3):
            for i in range(2):
                ag[(i, chain)].wait()
            s = hop_at[chain] + 1
            hop_at[chain] = s
            if s <= PLANE - 2:
                for i in range(2):
                    ag[(i, chain)] = ag_hop(i, chain, offs[chain], s)

        @functools.partial(pl.run_scoped, exit_sem=pltpu.SemaphoreType.REGULAR)
        def _(exit_sem):
            for nbr in neighbors:
                pl.semaphore_signal(exit_sem, inc=1, device_id=(nbr,),
                                    device_id_type=pl.DeviceIdType.MESH)
            pl.semaphore_wait(exit_sem, len(neighbors))

    return pl.pallas_call(
        body,
        out_shape=jax.ShapeDtypeStruct((m, n), jnp.bfloat16),
        in_specs=[
            pl.BlockSpec(memory_space=pltpu.VMEM),
            pl.BlockSpec(memory_space=pltpu.VMEM),
        ],
        out_specs=pl.BlockSpec(memory_space=pltpu.VMEM),
        scratch_shapes=[
            pltpu.VMEM((m, hn), jnp.bfloat16),
            pltpu.VMEM((m, hn), jnp.bfloat16),
            pltpu.VMEM((PLANE - 1, h2, hn), jnp.bfloat16),
            pltpu.VMEM((PLANE - 1, h2, hn), jnp.bfloat16),
            pltpu.VMEM((PLANE - 1, h2, hn), jnp.bfloat16),
            pltpu.VMEM((PLANE - 1, h2, hn), jnp.bfloat16),
            pltpu.VMEM((4, h4, hn), jnp.bfloat16),
            pltpu.VMEM((2, h4, hn), jnp.bfloat16),
            pltpu.SemaphoreType.DMA((6,)),
            pltpu.SemaphoreType.DMA((6,)),
            pltpu.SemaphoreType.DMA((6,)),
            pltpu.SemaphoreType.DMA((6,)),
            pltpu.SemaphoreType.DMA((12,)),
            pltpu.SemaphoreType.DMA((12,)),
            pltpu.SemaphoreType.DMA((12,)),
            pltpu.SemaphoreType.DMA((12,)),
            pltpu.SemaphoreType.DMA((12,)),
            pltpu.SemaphoreType.DMA((12,)),
        ],
        compiler_params=pltpu.CompilerParams(collective_id=0),
    )(A, B)
